# baseline (speedup 1.0000x reference)
import functools

import jax
import jax.numpy as jnp
from jax import lax
from jax.experimental import pallas as pl
from jax.experimental.pallas import tpu as pltpu
from jax.experimental.pallas import tpu_sc as plsc

_B = 16384
_L = 16
_info = plsc.get_sparse_core_info()
_NC = _info.num_cores
_NS = _info.num_subcores
_NW = _NC * _NS
_BW = _B // _NW
_E = _BW * 3
_CHUNK = 128
_NCH = _E // _CHUNK
_GPC = _CHUNK // _L
_TTC_PAD = 32


def _sc_body(lv_hbm, av_hbm, lin_hbm, ang_hbm, ttc_hbm, cam_hbm, sen_hbm,
             vel_out, ttc_out,
             cam_v, sen_v, idx_v, lin_vals, ang_vals, loc_lin, loc_ang,
             out_v, ttc_tab, ttc_val, sem):
  wid = lax.axis_index("s") * _NC + lax.axis_index("c")
  base = wid * _BW

  pltpu.sync_copy(cam_hbm.at[pl.ds(base, _BW)], cam_v)
  pltpu.sync_copy(sen_hbm.at[pl.ds(base, _BW)], sen_v)
  pltpu.sync_copy(lv_hbm.at[pl.ds(base * 3, _E)], loc_lin)
  pltpu.sync_copy(av_hbm.at[pl.ds(base * 3, _E)], loc_ang)
  pltpu.sync_copy(ttc_hbm, ttc_tab)

  lane = lax.iota(jnp.int32, _L)

  copies = []
  for t in range(_NCH):
    row = idx_v.at[t]
    for g in range(_GPC):
      e = lane + (t * _CHUNK + g * _L)
      k = e // 3
      c = e - k * 3
      cams = plsc.load_gather(cam_v, [k])
      row[pl.ds(g * _L, _L)] = cams * 3 + c
    s = t * _CHUNK
    copies.append(pltpu.async_copy(
        lin_hbm.at[row], lin_vals.at[pl.ds(s, _CHUNK)], sem))
    copies.append(pltpu.async_copy(
        ang_hbm.at[row], ang_vals.at[pl.ds(s, _CHUNK)], sem))

  def ttc_body(i, carry):
    sv = sen_v[pl.ds(i * _L, _L)]
    ttc_val[pl.ds(i * _L, _L)] = plsc.load_gather(ttc_tab, [sv])
    return carry
  lax.fori_loop(0, _BW // _L, ttc_body, 0)

  for cpy in copies:
    cpy.wait()

  for t in range(_NCH):
    for g in range(_GPC):
      off = t * _CHUNK + g * _L
      e = lane + off
      k = e // 3
      c = e - k * 3
      dst = k * 6 + c
      sl = loc_lin[pl.ds(off, _L)] + lin_vals[pl.ds(off, _L)]
      sa = loc_ang[pl.ds(off, _L)] + ang_vals[pl.ds(off, _L)]
      plsc.store_scatter(out_v, [dst], sl)
      plsc.store_scatter(out_v, [dst + 3], sa)

  pltpu.sync_copy(out_v, vel_out.at[pl.ds(base * 6, _BW * 6)])
  pltpu.sync_copy(ttc_val, ttc_out.at[pl.ds(base, _BW)])


_launch = functools.partial(
    pl.kernel,
    mesh=plsc.VectorSubcoreMesh(core_axis_name="c", subcore_axis_name="s"),
    compiler_params=pltpu.CompilerParams(needs_layout_passes=False,
                                         use_tc_tiling_on_sc=False),
    out_type=(
        jax.ShapeDtypeStruct((_B * 6,), jnp.float32),
        jax.ShapeDtypeStruct((_B,), jnp.float32),
    ),
    scratch_types=[
        pltpu.VMEM((_BW,), jnp.int32),
        pltpu.VMEM((_BW,), jnp.int32),
        pltpu.VMEM((_NCH, _CHUNK), jnp.int32),
        pltpu.VMEM((_E,), jnp.float32),
        pltpu.VMEM((_E,), jnp.float32),
        pltpu.VMEM((_E,), jnp.float32),
        pltpu.VMEM((_E,), jnp.float32),
        pltpu.VMEM((_BW * 6,), jnp.float32),
        pltpu.VMEM((_TTC_PAD,), jnp.float32),
        pltpu.VMEM((_BW,), jnp.float32),
        pltpu.SemaphoreType.DMA,
    ],
)(_sc_body)


def kernel(linear_velocities_local, angular_velocities_local,
           linear_velocity_adjustment, angular_velocity_adjustment,
           time_to_center_pixel_adjustment, cam_idx, sensor_idx):
  ttc_pad = jnp.pad(time_to_center_pixel_adjustment,
                    (0, _TTC_PAD - time_to_center_pixel_adjustment.shape[0]))
  vel_flat, ttc = _launch(
      linear_velocities_local.reshape(-1),
      angular_velocities_local.reshape(-1),
      linear_velocity_adjustment.reshape(-1),
      angular_velocity_adjustment.reshape(-1),
      ttc_pad,
      cam_idx.astype(jnp.int32),
      sensor_idx.astype(jnp.int32),
  )
  return vel_flat.reshape(_B, 6), ttc


# speedup vs baseline: 1.0012x; 1.0012x over previous
import functools

import jax
import jax.numpy as jnp
from jax import lax
from jax.experimental import pallas as pl
from jax.experimental.pallas import tpu as pltpu
from jax.experimental.pallas import tpu_sc as plsc

_B = 16384
_L = 16
_info = plsc.get_sparse_core_info()
_NC = _info.num_cores
_NS = _info.num_subcores
_NW = _NC * _NS
_BW = _B // _NW
_E = _BW * 3
_CHUNK = 128
_NCH = _E // _CHUNK
_GPC = _CHUNK // _L
_TTC_PAD = 32


def _sc_body(lv_hbm, av_hbm, lin_hbm, ang_hbm, ttc_hbm, cam_hbm, sen_hbm,
             vel_out, ttc_out,
             cam_v, sen_v, idx_v, lin_vals, ang_vals, loc_lin, loc_ang,
             out_v, ttc_tab, ttc_val, sem):
  wid = lax.axis_index("s") * _NC + lax.axis_index("c")
  base = wid * _BW

  pltpu.sync_copy(cam_hbm.at[pl.ds(base, _BW)], cam_v)
  pltpu.sync_copy(sen_hbm.at[pl.ds(base, _BW)], sen_v)
  pltpu.sync_copy(lv_hbm.at[pl.ds(base * 3, _E)], loc_lin)
  pltpu.sync_copy(av_hbm.at[pl.ds(base * 3, _E)], loc_ang)
  pltpu.sync_copy(ttc_hbm, ttc_tab)

  lane = lax.iota(jnp.int32, _L)

  copies = []
  for t in range(_NCH):
    row = idx_v.at[t]
    for g in range(_GPC):
      e = lane + (t * _CHUNK + g * _L)
      k = e // 3
      c = e - k * 3
      cams = plsc.load_gather(cam_v, [k])
      row[pl.ds(g * _L, _L)] = cams * 3 + c
    s = t * _CHUNK
    copies.append(pltpu.async_copy(
        lin_hbm.at[row], lin_vals.at[pl.ds(s, _CHUNK)], sem))
    copies.append(pltpu.async_copy(
        ang_hbm.at[row], ang_vals.at[pl.ds(s, _CHUNK)], sem))

  def ttc_body(i, carry):
    sv = sen_v[pl.ds(i * _L, _L)]
    ttc_val[pl.ds(i * _L, _L)] = plsc.load_gather(ttc_tab, [sv])
    return carry
  lax.fori_loop(0, _BW // _L, ttc_body, 0)

  for cpy in copies:
    cpy.wait()

  for t in range(_NCH):
    for g in range(_GPC):
      off = t * _CHUNK + g * _L
      e = lane + off
      k = e // 3
      c = e - k * 3
      dst = k * 6 + c
      sl = loc_lin[pl.ds(off, _L)] + lin_vals[pl.ds(off, _L)]
      sa = loc_ang[pl.ds(off, _L)] + ang_vals[pl.ds(off, _L)]
      plsc.store_scatter(out_v, [dst], sl)
      plsc.store_scatter(out_v, [dst + 3], sa)

  pltpu.sync_copy(out_v, vel_out.at[pl.ds(base * 6, _BW * 6)])
  pltpu.sync_copy(ttc_val, ttc_out.at[pl.ds(base, _BW)])


_launch = functools.partial(
    pl.kernel,
    mesh=plsc.VectorSubcoreMesh(core_axis_name="c", subcore_axis_name="s"),
    compiler_params=pltpu.CompilerParams(needs_layout_passes=False),
    out_type=(
        jax.ShapeDtypeStruct((_B * 6,), jnp.float32),
        jax.ShapeDtypeStruct((_B,), jnp.float32),
    ),
    scratch_types=[
        pltpu.VMEM((_BW,), jnp.int32),
        pltpu.VMEM((_BW,), jnp.int32),
        pltpu.VMEM((_NCH, _CHUNK), jnp.int32),
        pltpu.VMEM((_E,), jnp.float32),
        pltpu.VMEM((_E,), jnp.float32),
        pltpu.VMEM((_E,), jnp.float32),
        pltpu.VMEM((_E,), jnp.float32),
        pltpu.VMEM((_BW * 6,), jnp.float32),
        pltpu.VMEM((_TTC_PAD,), jnp.float32),
        pltpu.VMEM((_BW,), jnp.float32),
        pltpu.SemaphoreType.DMA,
    ],
)(_sc_body)


def kernel(linear_velocities_local, angular_velocities_local,
           linear_velocity_adjustment, angular_velocity_adjustment,
           time_to_center_pixel_adjustment, cam_idx, sensor_idx):
  ttc_pad = jnp.pad(time_to_center_pixel_adjustment,
                    (0, _TTC_PAD - time_to_center_pixel_adjustment.shape[0]))
  vel_flat, ttc = _launch(
      linear_velocities_local.reshape(-1),
      angular_velocities_local.reshape(-1),
      linear_velocity_adjustment.reshape(-1),
      angular_velocity_adjustment.reshape(-1),
      ttc_pad,
      cam_idx.astype(jnp.int32),
      sensor_idx.astype(jnp.int32),
  )
  return vel_flat.reshape(_B, 6), ttc


# speedup vs baseline: 44.1712x; 44.1169x over previous
import functools

import jax
import jax.numpy as jnp
from jax import lax
from jax.experimental import pallas as pl
from jax.experimental.pallas import tpu as pltpu
from jax.experimental.pallas import tpu_sc as plsc

_B = 16384
_L = 16
_info = plsc.get_sparse_core_info()
_NC = _info.num_cores
_NS = _info.num_subcores
_NW = _NC * _NS
_BW = _B // _NW
_E = _BW * 3
_CHUNK = 128
_NU = _BW // _CHUNK
_GPC = _CHUNK // _L
_NCH = 3 * _NU
_TTC_PAD = 32


def _sc_body(lv_hbm, av_hbm, lin_hbm, ang_hbm, ttc_hbm, cam_hbm, sen_hbm,
             vel_out, ttc_out,
             cam_v, sen_v, idx_v, lin_vals, ang_vals, loc_lin, loc_ang,
             out_v, ttc_tab, ttc_val, sem):
  wid = lax.axis_index("s") * _NC + lax.axis_index("c")
  base = wid * _BW

  for u in range(_NU):
    pltpu.sync_copy(cam_hbm.at[pl.ds(base + u * _CHUNK, _CHUNK)], cam_v.at[u])
  pltpu.sync_copy(sen_hbm.at[pl.ds(base, _BW)], sen_v)
  pltpu.sync_copy(lv_hbm.at[pl.ds(base * 3, _E)], loc_lin)
  pltpu.sync_copy(av_hbm.at[pl.ds(base * 3, _E)], loc_ang)
  pltpu.sync_copy(ttc_hbm, ttc_tab)

  lin_flat = lin_hbm.at[0]
  ang_flat = ang_hbm.at[0]

  copies = []
  for u in range(_NU):
    for g in range(_GPC):
      cams = cam_v.at[u][pl.ds(g * _L, _L)]
      row = idx_v.at[3 * u]
      row[pl.ds(g * _L, _L)] = cams
      row = idx_v.at[3 * u + 1]
      row[pl.ds(g * _L, _L)] = cams + 1000000
      row = idx_v.at[3 * u + 2]
      row[pl.ds(g * _L, _L)] = cams + 2000000
    for c in range(3):
      t = 3 * u + c
      copies.append(pltpu.async_copy(
          lin_flat.at[idx_v.at[t]], lin_vals.at[pl.ds(t * _CHUNK, _CHUNK)],
          sem))
      copies.append(pltpu.async_copy(
          ang_flat.at[idx_v.at[t]], ang_vals.at[pl.ds(t * _CHUNK, _CHUNK)],
          sem))

  def ttc_body(i, carry):
    sv = sen_v[pl.ds(i * _L, _L)]
    ttc_val[pl.ds(i * _L, _L)] = plsc.load_gather(ttc_tab, [sv])
    return carry
  lax.fori_loop(0, _BW // _L, ttc_body, 0)

  for cpy in copies:
    cpy.wait()

  lane = lax.iota(jnp.int32, _L)

  for u in range(_NU):
    for g in range(_GPC):
      k = lane + (u * _CHUNK + g * _L)
      for c in range(3):
        off = (3 * u + c) * _CHUNK + g * _L
        sl = plsc.load_gather(loc_lin, [k * 3 + c]) + \
            lin_vals[pl.ds(off, _L)]
        sa = plsc.load_gather(loc_ang, [k * 3 + c]) + \
            ang_vals[pl.ds(off, _L)]
        plsc.store_scatter(out_v, [k * 6 + c], sl)
        plsc.store_scatter(out_v, [k * 6 + c + 3], sa)

  pltpu.sync_copy(out_v, vel_out.at[pl.ds(base * 6, _BW * 6)])
  pltpu.sync_copy(ttc_val, ttc_out.at[pl.ds(base, _BW)])


_launch = functools.partial(
    pl.kernel,
    mesh=plsc.VectorSubcoreMesh(core_axis_name="c", subcore_axis_name="s"),
    compiler_params=pltpu.CompilerParams(needs_layout_passes=False,
                                         use_tc_tiling_on_sc=False),
    out_type=(
        jax.ShapeDtypeStruct((_B * 6,), jnp.float32),
        jax.ShapeDtypeStruct((_B,), jnp.float32),
    ),
    scratch_types=[
        pltpu.VMEM((_NU, _CHUNK), jnp.int32),
        pltpu.VMEM((_BW,), jnp.int32),
        pltpu.VMEM((_NCH, _CHUNK), jnp.int32),
        pltpu.VMEM((_E,), jnp.float32),
        pltpu.VMEM((_E,), jnp.float32),
        pltpu.VMEM((_E,), jnp.float32),
        pltpu.VMEM((_E,), jnp.float32),
        pltpu.VMEM((_BW * 6,), jnp.float32),
        pltpu.VMEM((_TTC_PAD,), jnp.float32),
        pltpu.VMEM((_BW,), jnp.float32),
        pltpu.SemaphoreType.DMA,
    ],
)(_sc_body)


def kernel(linear_velocities_local, angular_velocities_local,
           linear_velocity_adjustment, angular_velocity_adjustment,
           time_to_center_pixel_adjustment, cam_idx, sensor_idx):
  ttc_pad = jnp.pad(time_to_center_pixel_adjustment,
                    (0, _TTC_PAD - time_to_center_pixel_adjustment.shape[0]))
  vel_flat, ttc = _launch(
      linear_velocities_local.reshape(-1),
      angular_velocities_local.reshape(-1),
      jnp.swapaxes(linear_velocity_adjustment, 0, 1),
      jnp.swapaxes(angular_velocity_adjustment, 0, 1),
      ttc_pad,
      cam_idx.astype(jnp.int32),
      sensor_idx.astype(jnp.int32),
  )
  return vel_flat.reshape(_B, 6), ttc


# speedup vs baseline: 59.4451x; 1.3458x over previous
import functools

import jax
import jax.numpy as jnp
from jax import lax
from jax.experimental import pallas as pl
from jax.experimental.pallas import tpu as pltpu
from jax.experimental.pallas import tpu_sc as plsc

_B = 16384
_V = 1000000
_L = 16
_info = plsc.get_sparse_core_info()
_NC = _info.num_cores
_NS = _info.num_subcores
_NW = _NC * _NS
_BW = _B // _NW
_E = _BW * 3
_CHUNK = 128
_NU = _BW // _CHUNK
_GPC = _CHUNK // _L
_NCH = 3 * _NU
_GPW = _BW // _L
_TTC_PAD = 32


def _sc_body(lv_hbm, av_hbm, lin_hbm, ang_hbm, ttc_hbm, cam_hbm, sen_hbm,
             vel_out, ttc_out,
             cam_v, sen_v, idx_v, lin_vals, ang_vals, loc_lin, loc_ang,
             out_v, ttc_tab, ttc_val, sem):
  wid = lax.axis_index("s") * _NC + lax.axis_index("c")
  base = wid * _BW

  for u in range(_NU):
    pltpu.sync_copy(cam_hbm.at[pl.ds(base + u * _CHUNK, _CHUNK)], cam_v.at[u])
  pltpu.sync_copy(sen_hbm.at[pl.ds(base, _BW)], sen_v)
  for c in range(3):
    pltpu.sync_copy(lv_hbm.at[c].at[pl.ds(base, _BW)], loc_lin.at[c])
    pltpu.sync_copy(av_hbm.at[c].at[pl.ds(base, _BW)], loc_ang.at[c])
  pltpu.sync_copy(ttc_hbm, ttc_tab)

  lin_flat = lin_hbm.at[0]
  ang_flat = ang_hbm.at[0]

  copies = []
  for u in range(_NU):
    for g in range(_GPC):
      cams = cam_v.at[u][pl.ds(g * _L, _L)]
      for c in range(3):
        row = idx_v.at[c * _NU + u]
        row[pl.ds(g * _L, _L)] = cams + c * _V
    for c in range(3):
      t = c * _NU + u
      copies.append(pltpu.async_copy(
          lin_flat.at[idx_v.at[t]], lin_vals.at[pl.ds(t * _CHUNK, _CHUNK)],
          sem))
      copies.append(pltpu.async_copy(
          ang_flat.at[idx_v.at[t]], ang_vals.at[pl.ds(t * _CHUNK, _CHUNK)],
          sem))

  def ttc_body(i, carry):
    sv = sen_v[pl.ds(i * _L, _L)]
    ttc_val[pl.ds(i * _L, _L)] = plsc.load_gather(ttc_tab, [sv])
    return carry
  lax.fori_loop(0, _GPW, ttc_body, 0)

  for cpy in copies:
    cpy.wait()

  for c in range(3):
    for g in range(_GPW):
      o = c * _BW + g * _L
      sl = loc_lin.at[c][pl.ds(g * _L, _L)] + lin_vals[pl.ds(o, _L)]
      sa = loc_ang.at[c][pl.ds(g * _L, _L)] + ang_vals[pl.ds(o, _L)]
      out_v[pl.ds(o, _L)] = sl
      out_v[pl.ds(o + 3 * _BW, _L)] = sa

  for comp in range(6):
    pltpu.sync_copy(out_v.at[pl.ds(comp * _BW, _BW)],
                    vel_out.at[pl.ds(comp * _B + base, _BW)])
  pltpu.sync_copy(ttc_val, ttc_out.at[pl.ds(base, _BW)])


_launch = functools.partial(
    pl.kernel,
    mesh=plsc.VectorSubcoreMesh(core_axis_name="c", subcore_axis_name="s"),
    compiler_params=pltpu.CompilerParams(needs_layout_passes=False,
                                         use_tc_tiling_on_sc=False),
    out_type=(
        jax.ShapeDtypeStruct((_B * 6,), jnp.float32),
        jax.ShapeDtypeStruct((_B,), jnp.float32),
    ),
    scratch_types=[
        pltpu.VMEM((_NU, _CHUNK), jnp.int32),
        pltpu.VMEM((_BW,), jnp.int32),
        pltpu.VMEM((_NCH, _CHUNK), jnp.int32),
        pltpu.VMEM((_E,), jnp.float32),
        pltpu.VMEM((_E,), jnp.float32),
        pltpu.VMEM((3, _BW), jnp.float32),
        pltpu.VMEM((3, _BW), jnp.float32),
        pltpu.VMEM((_BW * 6,), jnp.float32),
        pltpu.VMEM((_TTC_PAD,), jnp.float32),
        pltpu.VMEM((_BW,), jnp.float32),
        pltpu.SemaphoreType.DMA,
    ],
)(_sc_body)


def kernel(linear_velocities_local, angular_velocities_local,
           linear_velocity_adjustment, angular_velocity_adjustment,
           time_to_center_pixel_adjustment, cam_idx, sensor_idx):
  ttc_pad = jnp.pad(time_to_center_pixel_adjustment,
                    (0, _TTC_PAD - time_to_center_pixel_adjustment.shape[0]))
  vel_flat, ttc = _launch(
      jnp.swapaxes(linear_velocities_local, 0, 1),
      jnp.swapaxes(angular_velocities_local, 0, 1),
      jnp.swapaxes(linear_velocity_adjustment, 0, 1),
      jnp.swapaxes(angular_velocity_adjustment, 0, 1),
      ttc_pad,
      cam_idx.astype(jnp.int32),
      sensor_idx.astype(jnp.int32),
  )
  return jnp.swapaxes(vel_flat.reshape(6, _B), 0, 1), ttc
